# baseline (speedup 1.0000x reference)
import jax
import jax.numpy as jnp
from jax.experimental import pallas as pl

T = 12
N = 307
C = 64
B = 8
NP = 384
BC = B * C
NW = T - 2
NG = 3


def _body(xt, tec, seb, at, wt, bc, out):
    f32 = jnp.float32
    i = pl.program_id(0)
    X0 = xt[i] + tec[i] + seb[...]
    X1 = xt[i + 1] + tec[i + 1] + seb[...]
    X2 = xt[i + 2] + tec[i + 2] + seb[...]

    def glu(y, wtj, bcj):
        parts = []
        for bi in range(B):
            yb = y[bi * C:(bi + 1) * C, :]
            t = jnp.dot(wtj, yb, preferred_element_type=f32) + bcj
            parts.append(t[:C] * jax.nn.sigmoid(t[C:]))
        return jnp.concatenate(parts, axis=0)

    h = None
    acc = None
    for j in range(NG):
        wtj = wt[j]
        bcj = bc[j]
        if j == 0:
            y = (jnp.dot(X0, at[0:NP, :], preferred_element_type=f32)
                 + jnp.dot(X1, at[NP:2 * NP, :], preferred_element_type=f32)
                 + jnp.dot(X2, at[2 * NP:3 * NP, :], preferred_element_type=f32))
        elif j == 1:
            y = jnp.dot(h, at[...], preferred_element_type=f32)
        else:
            y = jnp.dot(h, at[:, NP:2 * NP], preferred_element_type=f32)
        g = glu(y, wtj, bcj)
        if j < NG - 1:
            h = g
            c = g[:, NP:2 * NP]
        else:
            c = g
        acc = c if acc is None else jnp.maximum(acc, c)
    out[0] = acc


def kernel(x, A, temporal_emb, spatial_emb, W, b):
    xt = jnp.transpose(x, (1, 0, 3, 2)).reshape(T, BC, N)
    xt = jnp.pad(xt, ((0, 0), (0, 0), (0, NP - N)))

    tec = jnp.tile(temporal_emb.reshape(T, C), (1, B)).reshape(T, BC, 1)
    seb = jnp.tile(spatial_emb.reshape(N, C).T, (B, 1))
    seb = jnp.pad(seb, ((0, 0), (0, NP - N)))

    A4 = A.reshape(3, N, 3, N)
    Ap = jnp.pad(A4, ((0, 0), (0, NP - N), (0, 0), (0, NP - N)))
    AT = jnp.transpose(Ap.reshape(3 * NP, 3 * NP))

    WT = jnp.transpose(W, (0, 2, 1))
    bcol = b.reshape(NW * NG, 2 * C, 1)

    full = lambda shape: pl.BlockSpec(shape, lambda i: (0,) * len(shape))

    out = pl.pallas_call(
        _body,
        grid=(NW,),
        in_specs=[
            full((T, BC, NP)),
            full((T, BC, 1)),
            full((BC, NP)),
            full((3 * NP, 3 * NP)),
            pl.BlockSpec((NG, 2 * C, C), lambda i: (i, 0, 0)),
            pl.BlockSpec((NG, 2 * C, 1), lambda i: (i, 0, 0)),
        ],
        out_specs=pl.BlockSpec((1, BC, NP), lambda i: (i, 0, 0)),
        out_shape=jax.ShapeDtypeStruct((NW, BC, NP), jnp.float32),
    )(xt, tec, seb, AT, WT, bcol)

    o = out[:, :, :N].reshape(NW, B, C, N)
    return jnp.transpose(o, (1, 0, 3, 2))


# speedup vs baseline: 1.0069x; 1.0069x over previous
import jax
import jax.numpy as jnp
from jax.experimental import pallas as pl

T = 12
N = 307
C = 64
B = 8
NP = 384
BC = B * C
NW = T - 2
NG = 3


def _body(x0, x1, x2, tec, seb, at, wt, bc, out):
    f32 = jnp.float32
    i = pl.program_id(0)
    X0 = x0[0] + tec[i] + seb[...]
    X1 = x1[0] + tec[i + 1] + seb[...]
    X2 = x2[0] + tec[i + 2] + seb[...]

    def glu(y, wtj, bcj):
        parts = []
        for bi in range(B):
            yb = y[bi * C:(bi + 1) * C, :]
            t = jnp.dot(wtj, yb, preferred_element_type=f32) + bcj
            parts.append(t[:C] * jax.nn.sigmoid(t[C:]))
        return jnp.concatenate(parts, axis=0)

    h = None
    acc = None
    for j in range(NG):
        wtj = wt[j]
        bcj = bc[j]
        if j == 0:
            y = (jnp.dot(X0, at[0:NP, :], preferred_element_type=f32)
                 + jnp.dot(X1, at[NP:2 * NP, :], preferred_element_type=f32)
                 + jnp.dot(X2, at[2 * NP:3 * NP, :], preferred_element_type=f32))
        elif j == 1:
            y = jnp.dot(h, at[...], preferred_element_type=f32)
        else:
            y = jnp.dot(h, at[:, NP:2 * NP], preferred_element_type=f32)
        g = glu(y, wtj, bcj)
        if j < NG - 1:
            h = g
            c = g[:, NP:2 * NP]
        else:
            c = g
        acc = c if acc is None else jnp.maximum(acc, c)
    out[0] = acc


def kernel(x, A, temporal_emb, spatial_emb, W, b):
    xt = jnp.transpose(x, (1, 0, 3, 2)).reshape(T, BC, N)
    xt = jnp.pad(xt, ((0, 0), (0, 0), (0, NP - N)))

    tec = jnp.tile(temporal_emb.reshape(T, C), (1, B)).reshape(T, BC, 1)
    seb = jnp.tile(spatial_emb.reshape(N, C).T, (B, 1))
    seb = jnp.pad(seb, ((0, 0), (0, NP - N)))

    A4 = A.reshape(3, N, 3, N)
    Ap = jnp.pad(A4, ((0, 0), (0, NP - N), (0, 0), (0, NP - N)))
    AT = jnp.transpose(Ap.reshape(3 * NP, 3 * NP))

    WT = jnp.transpose(W, (0, 2, 1))
    bcol = b.reshape(NW * NG, 2 * C, 1)

    full = lambda shape: pl.BlockSpec(shape, lambda i: (0,) * len(shape))

    out = pl.pallas_call(
        _body,
        grid=(NW,),
        in_specs=[
            pl.BlockSpec((1, BC, NP), lambda i: (i, 0, 0)),
            pl.BlockSpec((1, BC, NP), lambda i: (i + 1, 0, 0)),
            pl.BlockSpec((1, BC, NP), lambda i: (i + 2, 0, 0)),
            full((T, BC, 1)),
            full((BC, NP)),
            full((3 * NP, 3 * NP)),
            pl.BlockSpec((NG, 2 * C, C), lambda i: (i, 0, 0)),
            pl.BlockSpec((NG, 2 * C, 1), lambda i: (i, 0, 0)),
        ],
        out_specs=pl.BlockSpec((1, BC, NP), lambda i: (i, 0, 0)),
        out_shape=jax.ShapeDtypeStruct((NW, BC, NP), jnp.float32),
    )(xt, xt, xt, tec, seb, AT, WT, bcol)

    o = out[:, :, :N].reshape(NW, B, C, N)
    return jnp.transpose(o, (1, 0, 3, 2))


# speedup vs baseline: 1.0152x; 1.0082x over previous
import jax
import jax.numpy as jnp
from jax.experimental import pallas as pl

T = 12
N = 307
C = 64
B = 8
NP = 384
BC = B * C
NW = T - 2
NG = 3


def _body(x0, x1, x2, tec, seb, at, wt, bc, out):
    f32 = jnp.float32
    i = pl.program_id(0)
    X0 = x0[0] + tec[i] + seb[...]
    X1 = x1[0] + tec[i + 1] + seb[...]
    X2 = x2[0] + tec[i + 2] + seb[...]

    def glu(y, wtj, bcj):
        parts = []
        for bi in range(B):
            yb = y[bi * C:(bi + 1) * C, :]
            t = jnp.dot(wtj, yb, preferred_element_type=f32) + bcj
            parts.append(t[:C] * jax.nn.sigmoid(t[C:]))
        return jnp.concatenate(parts, axis=0)

    h = None
    acc = None
    for j in range(NG):
        wtj = wt[j]
        bcj = bc[j]
        if j == 0:
            y = (jnp.dot(X0, at[0:NP, :], preferred_element_type=f32)
                 + jnp.dot(X1, at[NP:2 * NP, :], preferred_element_type=f32)
                 + jnp.dot(X2, at[2 * NP:3 * NP, :], preferred_element_type=f32))
        elif j == 1:
            y = jnp.dot(h, at[...], preferred_element_type=f32)
        else:
            y = jnp.dot(h, at[:, NP:2 * NP], preferred_element_type=f32)
        g = glu(y, wtj, bcj)
        if j < NG - 1:
            h = g
            c = g[:, NP:2 * NP]
        else:
            c = g
        acc = c if acc is None else jnp.maximum(acc, c)
    for bi in range(B):
        tb = jnp.transpose(acc[bi * C:(bi + 1) * C, :])
        out[bi, 0] = tb[:N, :]


def kernel(x, A, temporal_emb, spatial_emb, W, b):
    xt = jnp.transpose(x, (1, 0, 3, 2)).reshape(T, BC, N)
    xt = jnp.pad(xt, ((0, 0), (0, 0), (0, NP - N)))

    tec = jnp.tile(temporal_emb.reshape(T, C), (1, B)).reshape(T, BC, 1)
    seb = jnp.tile(spatial_emb.reshape(N, C).T, (B, 1))
    seb = jnp.pad(seb, ((0, 0), (0, NP - N)))

    A4 = A.reshape(3, N, 3, N)
    Ap = jnp.pad(A4, ((0, 0), (0, NP - N), (0, 0), (0, NP - N)))
    AT = jnp.transpose(Ap.reshape(3 * NP, 3 * NP))

    WT = jnp.transpose(W, (0, 2, 1))
    bcol = b.reshape(NW * NG, 2 * C, 1)

    full = lambda shape: pl.BlockSpec(shape, lambda i: (0,) * len(shape))

    out = pl.pallas_call(
        _body,
        grid=(NW,),
        in_specs=[
            pl.BlockSpec((1, BC, NP), lambda i: (i, 0, 0)),
            pl.BlockSpec((1, BC, NP), lambda i: (i + 1, 0, 0)),
            pl.BlockSpec((1, BC, NP), lambda i: (i + 2, 0, 0)),
            full((T, BC, 1)),
            full((BC, NP)),
            full((3 * NP, 3 * NP)),
            pl.BlockSpec((NG, 2 * C, C), lambda i: (i, 0, 0)),
            pl.BlockSpec((NG, 2 * C, 1), lambda i: (i, 0, 0)),
        ],
        out_specs=pl.BlockSpec((B, 1, N, C), lambda i: (0, i, 0, 0)),
        out_shape=jax.ShapeDtypeStruct((B, NW, N, C), jnp.float32),
    )(xt, xt, xt, tec, seb, AT, WT, bcol)

    return out


# speedup vs baseline: 1.0745x; 1.0584x over previous
import jax
import jax.numpy as jnp
from jax import lax
from jax.experimental import pallas as pl
from jax.experimental.pallas import tpu as pltpu

T = 12
N = 307
C = 64
B = 8
NP = 384
BC = B * C
NW = T - 2
NG = 3

_TDIMS = (((1,), (1,)), ((), ()))


def _body(xr, tec, seb, ap, wt, bc, out, xe):
    f32 = jnp.float32
    i = pl.program_id(0)

    def fill(t):
        for bi in range(B):
            slab = jnp.transpose(xr[bi * T + t])
            tecb = tec[t, bi * C:(bi + 1) * C]
            sebb = seb[bi * C:(bi + 1) * C, :N]
            v = jnp.pad(slab + tecb + sebb, ((0, 0), (0, NP - N)))
            xe[t, bi * C:(bi + 1) * C, :] = v

    @pl.when(i == 0)
    def _():
        fill(0)
        fill(1)
    fill(i + 2)

    X0 = xe[i]
    X1 = xe[i + 1]
    X2 = xe[i + 2]

    def glu(y, wtj, bcj):
        parts = []
        for bi in range(B):
            yb = y[bi * C:(bi + 1) * C, :]
            t = jnp.dot(wtj, yb, preferred_element_type=f32) + bcj
            parts.append(t[:C] * jax.nn.sigmoid(t[C:]))
        return jnp.concatenate(parts, axis=0)

    h = None
    acc = None
    for j in range(NG):
        wtj = jnp.transpose(wt[j])
        bcj = bc[j]
        if j == 0:
            y = (lax.dot_general(X0, ap[:, 0:NP], _TDIMS, preferred_element_type=f32)
                 + lax.dot_general(X1, ap[:, NP:2 * NP], _TDIMS, preferred_element_type=f32)
                 + lax.dot_general(X2, ap[:, 2 * NP:3 * NP], _TDIMS, preferred_element_type=f32))
        elif j == 1:
            y = lax.dot_general(h, ap[...], _TDIMS, preferred_element_type=f32)
        else:
            y = lax.dot_general(h, ap[NP:2 * NP, :], _TDIMS, preferred_element_type=f32)
        g = glu(y, wtj, bcj)
        if j < NG - 1:
            h = g
            c = g[:, NP:2 * NP]
        else:
            c = g
        acc = c if acc is None else jnp.maximum(acc, c)
    for bi in range(B):
        tb = jnp.transpose(acc[bi * C:(bi + 1) * C, :])
        out[bi, 0] = tb[:N, :]


def kernel(x, A, temporal_emb, spatial_emb, W, b):
    xr = x.reshape(B * T, N, C)

    tec = jnp.tile(temporal_emb.reshape(T, C), (1, B)).reshape(T, BC, 1)
    seb = jnp.tile(spatial_emb.reshape(N, C).T, (B, 1))
    seb = jnp.pad(seb, ((0, 0), (0, NP - N)))

    A4 = A.reshape(3, N, 3, N)
    Ap = jnp.pad(A4, ((0, 0), (0, NP - N), (0, 0), (0, NP - N)))
    Ap = Ap.reshape(3 * NP, 3 * NP)

    bcol = b.reshape(NW * NG, 2 * C, 1)

    full = lambda shape: pl.BlockSpec(shape, lambda i: (0,) * len(shape))

    out = pl.pallas_call(
        _body,
        grid=(NW,),
        in_specs=[
            full((B * T, N, C)),
            full((T, BC, 1)),
            full((BC, NP)),
            full((3 * NP, 3 * NP)),
            pl.BlockSpec((NG, C, 2 * C), lambda i: (i, 0, 0)),
            pl.BlockSpec((NG, 2 * C, 1), lambda i: (i, 0, 0)),
        ],
        out_specs=pl.BlockSpec((B, 1, N, C), lambda i: (0, i, 0, 0)),
        out_shape=jax.ShapeDtypeStruct((B, NW, N, C), jnp.float32),
        scratch_shapes=[pltpu.VMEM((T, BC, NP), jnp.float32)],
    )(xr, tec, seb, Ap, W, bcol)

    return out
